# baseline (speedup 1.0000x reference)
import functools

import jax
import jax.numpy as jnp
from jax import lax
from jax.experimental import pallas as pl
from jax.experimental.pallas import tpu as pltpu
from jax.experimental.pallas import tpu_sc as plsc

HEADS = 16
GS = 4
NG = HEADS // GS
QL = 2048
KL = 2048
HEAVY = 204
RECENT = 204
BQ = 512
QB = QL // BQ
NV = KL // 16

F32_MIN = float(jnp.finfo(jnp.float32).min)
_SUM_RECENT = float(RECENT * (RECENT + 1) // 2 + (QL - RECENT) * (RECENT + 1))
_WMAX = float(QL - RECENT - 1)
_DSCALE = GS / HEADS / (QL * (QL + 1) / 2.0)


def _sc_select_body(imp_hbm, heavy_hbm, vals_ref):
    c = lax.axis_index("c")
    s = lax.axis_index("s")
    head = c * 4 + s

    @pl.when(s < 4)
    def _():
        pltpu.sync_copy(imp_hbm.at[head], vals_ref)

        lane = lax.broadcasted_iota(jnp.int32, (16,), 0)
        zero = jnp.zeros((16,), jnp.int32)

        def _u(j):
            return lax.bitcast_convert_type(vals_ref[pl.ds(j * 16, 16)],
                                            jnp.int32)

        def _tot(acc):
            tot = acc[0]
            for i in range(1, 16):
                tot = tot + acc[i]
            return tot

        def count_ge(cand):
            def body(j, acc):
                return acc + jnp.where(_u(j) >= cand, 1, 0).astype(jnp.int32)

            return _tot(lax.fori_loop(0, NV, body, zero, unroll=8))

        def bit_body(i, p):
            cand = p | (jnp.int32(1) << (jnp.int32(30) - i))
            return jnp.where(count_ge(cand) >= HEAVY, cand, p)

        p = lax.fori_loop(0, 31, bit_body, jnp.int32(0))

        cnt_gt = count_ge(p + 1)
        need = HEAVY - cnt_gt

        def cnt_eq_lt(m):
            def body(j, acc):
                msk = (_u(j) == p) & ((lane + j * 16) < m)
                return acc + jnp.where(msk, 1, 0).astype(jnp.int32)

            return _tot(lax.fori_loop(0, NV, body, zero, unroll=8))

        def t_body(i, t):
            cand = t | (jnp.int32(1) << (jnp.int32(10) - i))
            return jnp.where(cnt_eq_lt(cand) < need, cand, t)

        t = lax.fori_loop(0, 11, t_body, jnp.int32(0))

        def wr(j, carry):
            u = _u(j)
            ik = lane + j * 16
            sel = (u > p) | ((u == p) & (ik <= t))
            vals_ref[pl.ds(j * 16, 16)] = jnp.where(sel, 1.0, 0.0)
            return carry

        lax.fori_loop(0, NV, wr, jnp.int32(0), unroll=8)
        pltpu.sync_copy(vals_ref, heavy_hbm.at[head])


def _sc_select(imp8):
    mesh = plsc.VectorSubcoreMesh(core_axis_name="c", subcore_axis_name="s")
    f = functools.partial(
        pl.kernel,
        mesh=mesh,
        out_type=jax.ShapeDtypeStruct((2 * GS, KL), jnp.float32),
        scratch_types=[pltpu.VMEM((KL,), jnp.float32)],
    )(_sc_select_body)
    return f(imp8)


_TINY = pl.BlockSpec((1, 8, 128), lambda j, qb: (0, 0, 0))


def _imp_block(x):
    m = jnp.max(x, axis=-1, keepdims=True)
    e = jnp.exp(x - m)
    s = jnp.sum(e, axis=-1, keepdims=True)
    return jnp.sum(e / s, axis=1)[:, None, :]


def _a_call(gp, x, with_buffers):
    def body(*refs):
        x_ref, imp_ref = refs[0], refs[1]
        qb = pl.program_id(1)
        contrib = _imp_block(x_ref[...])

        @pl.when(qb == 0)
        def _():
            imp_ref[...] = contrib

        @pl.when(qb != 0)
        def _():
            imp_ref[...] += contrib

    out_specs = [pl.BlockSpec((GS, 1, KL), lambda j, qb: (j, 0, 0))]
    out_shape = [jax.ShapeDtypeStruct((2 * GS, 1, KL), jnp.float32)]
    if with_buffers:
        out_specs += [_TINY, pl.BlockSpec((1, 1), lambda j, qb: (0, 0))]
        out_shape += [
            jax.ShapeDtypeStruct((HEADS, QL, KL), jnp.float32),
            jax.ShapeDtypeStruct((1, 1), jnp.float32),
        ]

    outs = pl.pallas_call(
        body,
        grid=(2, QB),
        in_specs=[
            pl.BlockSpec((GS, BQ, KL), lambda j, qb, gp=gp: (2 * gp + j, qb, 0))
        ],
        out_specs=out_specs,
        out_shape=out_shape,
    )(x)
    return outs if with_buffers else outs[0]


def _c_call(gp, heavy8, mask_buf, dens_buf):
    def body(hv_ref, _m_in, d_in, mask_ref, dens_ref):
        j = pl.program_id(0)
        qb = pl.program_id(1)
        hv4 = hv_ref[:, 0, :]
        hg = jnp.max(hv4, axis=0, keepdims=True)
        hv = hg > 0.0
        iq = qb * BQ + lax.broadcasted_iota(jnp.int32, (BQ, 1), 0)
        ik = lax.broadcasted_iota(jnp.int32, (1, KL), 1)
        keep = (ik <= iq) & (hv | (ik >= iq - RECENT))
        blk = jnp.where(keep, 0.0, F32_MIN)
        mask_ref[...] = jnp.broadcast_to(blk[None], (GS, BQ, KL))

        @pl.when(qb == 0)
        def _():
            ikf = ik.astype(jnp.float32)
            w = jnp.maximum(0.0, _WMAX - ikf)
            count_g = _SUM_RECENT + jnp.sum(hg * w, axis=-1, keepdims=True)
            contrib = count_g * _DSCALE
            prev = d_in[...] if gp else jnp.zeros((1, 1), jnp.float32)
            dens_ref[...] = jnp.where(j == 0, prev + contrib,
                                      dens_ref[...] + contrib)

    return pl.pallas_call(
        body,
        grid=(2, QB),
        in_specs=[
            pl.BlockSpec((GS, 1, KL), lambda j, qb: (j, 0, 0)),
            _TINY,
            pl.BlockSpec((1, 1), lambda j, qb: (0, 0)),
        ],
        out_specs=[
            pl.BlockSpec((GS, BQ, KL), lambda j, qb, gp=gp: (2 * gp + j, qb, 0)),
            pl.BlockSpec((1, 1), lambda j, qb: (0, 0)),
        ],
        out_shape=[
            jax.ShapeDtypeStruct((HEADS, QL, KL), jnp.float32),
            jax.ShapeDtypeStruct((1, 1), jnp.float32),
        ],
        input_output_aliases={1: 0, 2: 1},
    )(heavy8, mask_buf, dens_buf)


def kernel(attn_weights, group_size):
    x = attn_weights.reshape(HEADS, QL, KL)

    imp01, mask_buf, dens_buf = _a_call(0, x, True)
    heavy01 = _sc_select(imp01.reshape(2 * GS, KL))
    imp23 = _a_call(1, x, False)
    mask_buf, dens_buf = _c_call(0, heavy01.reshape(2 * GS, 1, KL),
                                 mask_buf, dens_buf)
    heavy23 = _sc_select(imp23.reshape(2 * GS, KL))
    mask, dens = _c_call(1, heavy23.reshape(2 * GS, 1, KL),
                         mask_buf, dens_buf)

    density = dens.reshape(())
    density = density + (jnp.asarray(group_size) - GS).astype(jnp.float32) * 0.0
    return (mask.reshape(1, HEADS, QL, KL), density)


# speedup vs baseline: 1.0377x; 1.0377x over previous
import functools

import jax
import jax.numpy as jnp
from jax import lax
from jax.experimental import pallas as pl
from jax.experimental.pallas import tpu as pltpu
from jax.experimental.pallas import tpu_sc as plsc

HEADS = 16
GS = 4
NG = HEADS // GS
QL = 2048
KL = 2048
HEAVY = 204
RECENT = 204
BQ = 512
QB = QL // BQ
NV = KL // 16

F32_MIN = float(jnp.finfo(jnp.float32).min)
_SUM_RECENT = float(RECENT * (RECENT + 1) // 2 + (QL - RECENT) * (RECENT + 1))
_WMAX = float(QL - RECENT - 1)
_DSCALE = GS / HEADS / (QL * (QL + 1) / 2.0)


def _importance_kernel(x_ref, imp_ref):
    qb = pl.program_id(1)
    x = x_ref[...]
    m = jnp.max(x, axis=-1, keepdims=True)
    e = jnp.exp(x - m)
    s = jnp.sum(e, axis=-1, keepdims=True)
    contrib = jnp.sum(e / s, axis=1)[:, None, :]

    @pl.when(qb == 0)
    def _():
        imp_ref[...] = contrib

    @pl.when(qb != 0)
    def _():
        imp_ref[...] += contrib


def _sc_select_body(imp_hbm, heavy_hbm, vals_ref):
    c = lax.axis_index("c")
    s = lax.axis_index("s")
    head = c * 8 + s

    @pl.when(s < 8)
    def _():
        pltpu.sync_copy(imp_hbm.at[head], vals_ref)

        lane = lax.broadcasted_iota(jnp.int32, (16,), 0)
        zero = jnp.zeros((16,), jnp.int32)

        def _u(j):
            return lax.bitcast_convert_type(vals_ref[pl.ds(j * 16, 16)],
                                            jnp.int32)

        def _tot(acc):
            tot = acc[0]
            for i in range(1, 16):
                tot = tot + acc[i]
            return tot

        def count_ge(cand):
            def body(j, acc):
                return acc + jnp.where(_u(j) >= cand, 1, 0).astype(jnp.int32)

            return _tot(lax.fori_loop(0, NV, body, zero, unroll=8))

        def bit_body(i, p):
            cand = p | (jnp.int32(1) << (jnp.int32(30) - i))
            return jnp.where(count_ge(cand) >= HEAVY, cand, p)

        p = lax.fori_loop(0, 31, bit_body, jnp.int32(0))

        cnt_gt = count_ge(p + 1)
        need = HEAVY - cnt_gt

        def cnt_eq_lt(m):
            def body(j, acc):
                msk = (_u(j) == p) & ((lane + j * 16) < m)
                return acc + jnp.where(msk, 1, 0).astype(jnp.int32)

            return _tot(lax.fori_loop(0, NV, body, zero, unroll=8))

        def t_body(i, t):
            cand = t | (jnp.int32(1) << (jnp.int32(10) - i))
            return jnp.where(cnt_eq_lt(cand) < need, cand, t)

        t = lax.fori_loop(0, 11, t_body, jnp.int32(0))

        def wr(j, carry):
            u = _u(j)
            ik = lane + j * 16
            sel = (u > p) | ((u == p) & (ik <= t))
            vals_ref[pl.ds(j * 16, 16)] = jnp.where(sel, 1.0, 0.0)
            return carry

        lax.fori_loop(0, NV, wr, jnp.int32(0), unroll=8)
        pltpu.sync_copy(vals_ref, heavy_hbm.at[head])


def _sc_select(imp):
    mesh = plsc.VectorSubcoreMesh(core_axis_name="c", subcore_axis_name="s")
    f = functools.partial(
        pl.kernel,
        mesh=mesh,
        out_type=jax.ShapeDtypeStruct((HEADS, KL), jnp.float32),
        scratch_types=[pltpu.VMEM((KL,), jnp.float32)],
    )(_sc_select_body)
    return f(imp)


def _mask_kernel(hv_ref, mask_ref, dens_ref):
    g = pl.program_id(0)
    qb = pl.program_id(1)
    hv4 = hv_ref[:, 0, :]
    hg = jnp.max(hv4, axis=0, keepdims=True)
    hv = hg > 0.0

    @pl.when(qb == 0)
    def _():
        ikf = lax.broadcasted_iota(jnp.int32, (1, KL), 1).astype(jnp.float32)
        w = jnp.maximum(0.0, _WMAX - ikf)
        count_g = _SUM_RECENT + jnp.sum(hg * w, axis=-1, keepdims=True)
        contrib = count_g * _DSCALE
        dens_ref[...] = jnp.where(g == 0, contrib, dens_ref[...] + contrib)

    iq = qb * BQ + lax.broadcasted_iota(jnp.int32, (BQ, 1), 0)
    ik = lax.broadcasted_iota(jnp.int32, (1, KL), 1)
    keep = (ik <= iq) & (hv | (ik >= iq - RECENT))
    blk = jnp.where(keep, 0.0, F32_MIN)
    mask_ref[...] = jnp.broadcast_to(blk[None], (GS, BQ, KL))


def kernel(attn_weights, group_size):
    x = attn_weights.reshape(HEADS, QL, KL)

    imp = pl.pallas_call(
        _importance_kernel,
        grid=(NG, QB),
        in_specs=[pl.BlockSpec((GS, BQ, KL), lambda g, qb: (g, qb, 0))],
        out_specs=pl.BlockSpec((GS, 1, KL), lambda g, qb: (g, 0, 0)),
        out_shape=jax.ShapeDtypeStruct((HEADS, 1, KL), jnp.float32),
    )(x)

    heavy16 = _sc_select(imp.reshape(HEADS, KL))

    mask, dens = pl.pallas_call(
        _mask_kernel,
        grid=(NG, QB),
        in_specs=[pl.BlockSpec((GS, 1, KL), lambda g, qb: (g, 0, 0))],
        out_specs=[
            pl.BlockSpec((GS, BQ, KL), lambda g, qb: (g, qb, 0)),
            pl.BlockSpec((1, 1), lambda g, qb: (0, 0)),
        ],
        out_shape=[
            jax.ShapeDtypeStruct((HEADS, QL, KL), jnp.float32),
            jax.ShapeDtypeStruct((1, 1), jnp.float32),
        ],
    )(heavy16.reshape(HEADS, 1, KL))

    density = dens.reshape(())
    density = density + (jnp.asarray(group_size) - GS).astype(jnp.float32) * 0.0
    return (mask.reshape(1, HEADS, QL, KL), density)
